# Initial kernel scaffold; baseline (speedup 1.0000x reference)
import jax
import jax.numpy as jnp
from jax.experimental import pallas as pl


def kernel(x, edge_index, graph_pool, W1, a1s, a1d, b1, g1, be1, W2, a2s, a2d, b2, g2, be2, W3, a3s, a3d, b3):
    raise NotImplementedError("write your pallas kernel here")


# speedup vs baseline: 30.0446x; 30.0446x over previous
import functools

import jax
import jax.numpy as jnp
from jax import lax
from jax.experimental import pallas as pl
from jax.experimental.pallas import tpu as pltpu
from jax.experimental.pallas import tpu_sc as plsc

N = 10000
D = 128
G = 64
E = 320000
ET = E + N
NPAD = 10240
NW = 32
EPT = 10496
EPAD = EPT * NW
B = 128
NCHUNK = EPT // B
RPT = NPAD // 16

f32 = jnp.float32
i32 = jnp.int32


def _tc_pre_body(x_ref, w_ref, avs_ref, avd_ref, h_ref, asv_ref, adv_ref):
    h = jnp.dot(x_ref[...], w_ref[...], preferred_element_type=f32)
    h_ref[...] = h
    asv_ref[...] = jnp.dot(h, avs_ref[0])
    adv_ref[...] = jnp.dot(h, avd_ref[0])


def _tc_pre(x, w, avs, avd):
    return pl.pallas_call(
        _tc_pre_body,
        out_shape=(
            jax.ShapeDtypeStruct((NPAD, D), f32),
            jax.ShapeDtypeStruct((NPAD,), f32),
            jax.ShapeDtypeStruct((NPAD,), f32),
        ),
    )(x, w, avs, avd)


def _tc_mid_body(acc_ref, den_ref, b_ref, g_ref, be_ref, w_ref, avs_ref,
                 avd_ref, h_ref, asv_ref, adv_ref):
    num = acc_ref[0] + acc_ref[1]
    den = den_ref[0] + den_ref[1] + 1e-16
    o = num / den[:, None] + b_ref[...]
    mask = lax.broadcasted_iota(i32, (NPAD, 1), 0) < N
    om = jnp.where(mask, o, 0.0)
    inv_n = 1.0 / N
    mu = jnp.sum(om, axis=0) * inv_n
    var = jnp.sum(om * om, axis=0) * inv_n - mu * mu
    y = g_ref[...] * (o - mu) * lax.rsqrt(var + 1e-5) + be_ref[...]
    y = jnp.maximum(y, 0.0)
    y = jnp.where(mask, y, 0.0)
    h = jnp.dot(y, w_ref[...], preferred_element_type=f32)
    h_ref[...] = h
    asv_ref[...] = jnp.dot(h, avs_ref[0])
    adv_ref[...] = jnp.dot(h, avd_ref[0])


def _tc_mid(acc, den, b, g, be, w, avs, avd):
    return pl.pallas_call(
        _tc_mid_body,
        out_shape=(
            jax.ShapeDtypeStruct((NPAD, D), f32),
            jax.ShapeDtypeStruct((NPAD,), f32),
            jax.ShapeDtypeStruct((NPAD,), f32),
        ),
    )(acc, den, b, g, be, w, avs, avd)


def _tc_fin_body(acc_ref, den_ref, b_ref, gp_ref, pooled_ref, hn_ref):
    num = acc_ref[0] + acc_ref[1]
    den = den_ref[0] + den_ref[1] + 1e-16
    hn = num / den[:, None] + b_ref[...]
    mask = lax.broadcasted_iota(i32, (NPAD, 1), 0) < N
    hn = jnp.where(mask, hn, 0.0)
    hn_ref[...] = hn
    pooled_ref[...] = jnp.dot(gp_ref[...], hn, preferred_element_type=f32)


def _tc_fin(acc, den, b, gp):
    return pl.pallas_call(
        _tc_fin_body,
        out_shape=(
            jax.ShapeDtypeStruct((G, D), f32),
            jax.ShapeDtypeStruct((NPAD, D), f32),
        ),
    )(acc, den, b, gp)


def _edge_body(h_hbm, asv_hbm, adv_hbm, src_hbm, dst_hbm,
               acc_hbm, den_hbm,
               as_v, ad_v, src_v, dst_v, ex_v, rows_v,
               acc_sh, den_sh, sem1):
    c = lax.axis_index("c")
    s = lax.axis_index("s")
    wid = s * 2 + c
    base = wid * EPT

    pltpu.sync_copy(asv_hbm, as_v)
    pltpu.sync_copy(adv_hbm, ad_v)

    zv = jnp.zeros((16,), f32)

    def _zrow(i, _):
        for j in range(D // 16):
            rows_v[i, pl.ds(j * 16, 16)] = zv
        return 0

    lax.fori_loop(0, B, _zrow, 0, unroll=2)

    def _zex(i, _):
        ex_v[pl.ds(pl.multiple_of(i * 16, 16), 16)] = zv
        return 0

    lax.fori_loop(0, B // 16, _zex, 0)

    for k in range(RPT // B):
        pltpu.sync_copy(rows_v, acc_sh.at[pl.ds(s * RPT + k * B, B)])
        pltpu.sync_copy(ex_v, den_sh.at[pl.ds(s * RPT + k * B, B)])
    plsc.subcore_barrier()

    def _chunk(ci, _):
        off = base + ci * B
        pltpu.sync_copy(src_hbm.at[pl.ds(off, B)], src_v)
        pltpu.sync_copy(dst_hbm.at[pl.ds(off, B)], dst_v)
        gcp = pltpu.async_copy(h_hbm.at[src_v], rows_v, sem1)

        def _ex(j, _):
            sl = pl.ds(pl.multiple_of(j * 16, 16), 16)
            sv = plsc.load_gather(as_v, [src_v[sl]])
            dv = plsc.load_gather(ad_v, [dst_v[sl]])
            t = sv + dv
            t = jnp.where(t >= 0.0, t, 0.2 * t)
            ex_v[sl] = jnp.exp(t)
            return 0

        lax.fori_loop(0, B // 16, _ex, 0)
        gcp.wait()

        def _scale(g, _):
            exvec = ex_v[pl.ds(pl.multiple_of(g * 16, 16), 16)]
            for l in range(16):
                sc = exvec[l]
                e = g * 16 + l
                for f in range(D // 16):
                    sl = pl.ds(f * 16, 16)
                    rows_v[e, sl] = rows_v[e, sl] * sc
            return 0

        lax.fori_loop(0, B // 16, _scale, 0)

        pltpu.sync_copy(rows_v, acc_sh.at[dst_v], add=True)
        pltpu.sync_copy(ex_v, den_sh.at[dst_v], add=True)
        return 0

    lax.fori_loop(0, NCHUNK, _chunk, 0)
    plsc.subcore_barrier()

    pltpu.sync_copy(acc_sh.at[pl.ds(s * RPT, RPT)],
                    acc_hbm.at[c, pl.ds(s * RPT, RPT)])
    pltpu.sync_copy(den_sh.at[pl.ds(s * RPT, RPT)],
                    den_hbm.at[c, pl.ds(s * RPT, RPT)])


@functools.lru_cache(maxsize=1)
def _edge_pass_fn():
    return pl.kernel(
        _edge_body,
        out_type=(
            jax.ShapeDtypeStruct((2, NPAD, D), f32),
            jax.ShapeDtypeStruct((2, NPAD), f32),
        ),
        mesh=plsc.VectorSubcoreMesh(core_axis_name="c", subcore_axis_name="s"),
        compiler_params=pltpu.CompilerParams(needs_layout_passes=False),
        scratch_types=(
            pltpu.VMEM((NPAD,), f32),
            pltpu.VMEM((NPAD,), f32),
            pltpu.VMEM((B,), i32),
            pltpu.VMEM((B,), i32),
            pltpu.VMEM((B,), f32),
            pltpu.VMEM((B, D), f32),
            pltpu.VMEM_SHARED((NPAD, D), f32),
            pltpu.VMEM_SHARED((NPAD,), f32),
            pltpu.SemaphoreType.DMA,
        ),
    )


def _edge_pass(h, asv, adv, src, dst):
    return _edge_pass_fn()(h, asv, adv, src, dst)


def kernel(x, edge_index, graph_pool, W1, a1s, a1d, b1, g1, be1,
           W2, a2s, a2d, b2, g2, be2, W3, a3s, a3d, b3):
    ei = edge_index.astype(i32)
    loop = jnp.arange(N, dtype=i32)
    npad_e = EPAD - ET
    pad_idx = N + (jnp.arange(npad_e, dtype=i32) % 128)
    src = jnp.concatenate([ei[0], loop, pad_idx])
    dst = jnp.concatenate([ei[1], loop, pad_idx])

    x_p = jnp.zeros((NPAD, D), f32).at[:N].set(x)
    gp_p = jnp.zeros((G, NPAD), f32).at[:, :N].set(graph_pool)

    h1, as1, ad1 = _tc_pre(x_p, W1, a1s, a1d)
    acc1, den1 = _edge_pass(h1, as1, ad1, src, dst)
    h2, as2, ad2 = _tc_mid(acc1, den1, b1, g1, be1, W2, a2s, a2d)
    acc2, den2 = _edge_pass(h2, as2, ad2, src, dst)
    h3, as3, ad3 = _tc_mid(acc2, den2, b2, g2, be2, W3, a3s, a3d)
    acc3, den3 = _edge_pass(h3, as3, ad3, src, dst)
    pooled, hn = _tc_fin(acc3, den3, b3, gp_p)
    return (pooled, hn[:N])


# speedup vs baseline: 40.6165x; 1.3519x over previous
import functools

import jax
import jax.numpy as jnp
from jax import lax
from jax.experimental import pallas as pl
from jax.experimental.pallas import tpu as pltpu
from jax.experimental.pallas import tpu_sc as plsc

N = 10000
D = 128
G = 64
E = 320000
ET = E + N
NPAD = 10240
NW = 32
B = 64
NCHUNK = 162
EPT = NCHUNK * B
EPAD = EPT * NW
RPT = NPAD // 16

f32 = jnp.float32
i32 = jnp.int32


def _tc_pre_body(x_ref, w_ref, avs_ref, avd_ref, h_ref, asv_ref, adv_ref):
    h = jnp.dot(x_ref[...], w_ref[...], preferred_element_type=f32)
    h_ref[...] = h
    asv_ref[...] = jnp.dot(h, avs_ref[0])
    adv_ref[...] = jnp.dot(h, avd_ref[0])


def _tc_pre(x, w, avs, avd):
    return pl.pallas_call(
        _tc_pre_body,
        out_shape=(
            jax.ShapeDtypeStruct((NPAD, D), f32),
            jax.ShapeDtypeStruct((NPAD,), f32),
            jax.ShapeDtypeStruct((NPAD,), f32),
        ),
    )(x, w, avs, avd)


def _tc_mid_body(acc_ref, den_ref, b_ref, g_ref, be_ref, w_ref, avs_ref,
                 avd_ref, h_ref, asv_ref, adv_ref):
    num = acc_ref[0] + acc_ref[1]
    den = den_ref[0] + den_ref[1] + 1e-16
    o = num / den[:, None] + b_ref[...]
    mask = lax.broadcasted_iota(i32, (NPAD, 1), 0) < N
    om = jnp.where(mask, o, 0.0)
    inv_n = 1.0 / N
    mu = jnp.sum(om, axis=0) * inv_n
    var = jnp.sum(om * om, axis=0) * inv_n - mu * mu
    y = g_ref[...] * (o - mu) * lax.rsqrt(var + 1e-5) + be_ref[...]
    y = jnp.maximum(y, 0.0)
    y = jnp.where(mask, y, 0.0)
    h = jnp.dot(y, w_ref[...], preferred_element_type=f32)
    h_ref[...] = h
    asv_ref[...] = jnp.dot(h, avs_ref[0])
    adv_ref[...] = jnp.dot(h, avd_ref[0])


def _tc_mid(acc, den, b, g, be, w, avs, avd):
    return pl.pallas_call(
        _tc_mid_body,
        out_shape=(
            jax.ShapeDtypeStruct((NPAD, D), f32),
            jax.ShapeDtypeStruct((NPAD,), f32),
            jax.ShapeDtypeStruct((NPAD,), f32),
        ),
    )(acc, den, b, g, be, w, avs, avd)


def _tc_fin_body(acc_ref, den_ref, b_ref, gp_ref, pooled_ref, hn_ref):
    num = acc_ref[0] + acc_ref[1]
    den = den_ref[0] + den_ref[1] + 1e-16
    hn = num / den[:, None] + b_ref[...]
    mask = lax.broadcasted_iota(i32, (NPAD, 1), 0) < N
    hn = jnp.where(mask, hn, 0.0)
    hn_ref[...] = hn
    pooled_ref[...] = jnp.dot(gp_ref[...], hn, preferred_element_type=f32)


def _tc_fin(acc, den, b, gp):
    return pl.pallas_call(
        _tc_fin_body,
        out_shape=(
            jax.ShapeDtypeStruct((G, D), f32),
            jax.ShapeDtypeStruct((NPAD, D), f32),
        ),
    )(acc, den, b, gp)


def _edge_body(h_hbm, asv_hbm, adv_hbm, src_hbm, dst_hbm,
               acc_hbm, den_hbm,
               as_v, ad_v,
               sr0, sr1, sr2, ds0, ds1, ds2, ex0, ex1, ex2,
               rw0, rw1, rw2,
               acc_sh, den_sh,
               gs0, gs1, gs2, ss0, ss1, ss2, es0, es1, es2):
    c = lax.axis_index("c")
    s = lax.axis_index("s")
    wid = s * 2 + c
    base = wid * EPT

    src_v = (sr0, sr1, sr2)
    dst_v = (ds0, ds1, ds2)
    ex_v = (ex0, ex1, ex2)
    rows_v = (rw0, rw1, rw2)
    gsem = (gs0, gs1, gs2)
    ssem = (ss0, ss1, ss2)
    esem = (es0, es1, es2)

    pltpu.sync_copy(asv_hbm, as_v)
    pltpu.sync_copy(adv_hbm, ad_v)

    zv = jnp.zeros((16,), f32)

    def _zrow(i, _):
        for j in range(D // 16):
            rw0[i, pl.ds(j * 16, 16)] = zv
        return 0

    lax.fori_loop(0, B, _zrow, 0, unroll=2)

    def _zex(i, _):
        ex0[pl.ds(pl.multiple_of(i * 16, 16), 16)] = zv
        return 0

    lax.fori_loop(0, B // 16, _zex, 0)

    for k in range(RPT // B):
        pltpu.sync_copy(rw0, acc_sh.at[pl.ds(s * RPT + k * B, B)])
        pltpu.sync_copy(ex0, den_sh.at[pl.ds(s * RPT + k * B, B)])
    plsc.subcore_barrier()

    def _prefetch(b, ci):
        off = base + ci * B
        pltpu.sync_copy(src_hbm.at[pl.ds(off, B)], src_v[b])
        pltpu.sync_copy(dst_hbm.at[pl.ds(off, B)], dst_v[b])
        pltpu.async_copy(h_hbm.at[src_v[b]], rows_v[b], gsem[b])

    def _wait_gathers(b):
        pltpu.make_async_copy(
            h_hbm.at[src_v[b]], rows_v[b], gsem[b]).wait()

    def _wait_scatters(b):
        pltpu.make_async_copy(
            rows_v[b], acc_sh.at[dst_v[b]], ssem[b]).wait()
        pltpu.make_async_copy(
            ex_v[b], den_sh.at[dst_v[b]], esem[b]).wait()

    _prefetch(0, 0)
    _prefetch(1, 1)

    def _chunk3(k, _):
        for j in range(3):
            ci = 3 * k + j
            nb = (j + 2) % 3
            srj, dsj, exj, rwj = src_v[j], dst_v[j], ex_v[j], rows_v[j]

            _wait_gathers(j)

            def _ex(t, _):
                sl = pl.ds(pl.multiple_of(t * 16, 16), 16)
                sv = plsc.load_gather(as_v, [srj[sl]])
                dv = plsc.load_gather(ad_v, [dsj[sl]])
                tt = sv + dv
                tt = jnp.where(tt >= 0.0, tt, 0.2 * tt)
                exj[sl] = jnp.exp(tt)
                return 0

            lax.fori_loop(0, B // 16, _ex, 0)

            def _scale(g, _):
                exvec = exj[pl.ds(pl.multiple_of(g * 16, 16), 16)]
                for l in range(16):
                    sc = exvec[l]
                    e = g * 16 + l
                    for f in range(D // 16):
                        sl = pl.ds(f * 16, 16)
                        rwj[e, sl] = rwj[e, sl] * sc
                return 0

            lax.fori_loop(0, B // 16, _scale, 0)

            pltpu.async_copy(rwj, acc_sh.at[dsj], ssem[j], add=True)
            pltpu.async_copy(exj, den_sh.at[dsj], esem[j], add=True)

            if j == 0:
                @pl.when(k > 0)
                def _():
                    _wait_scatters(nb)
                    _prefetch(nb, ci + 2)

                @pl.when(k == 0)
                def _():
                    _prefetch(nb, ci + 2)
            else:
                @pl.when(ci + 2 < NCHUNK)
                def _():
                    _wait_scatters(nb)
                    _prefetch(nb, ci + 2)
        return 0

    lax.fori_loop(0, NCHUNK // 3, _chunk3, 0)
    for b in range(3):
        _wait_scatters(b)
    plsc.subcore_barrier()

    pltpu.sync_copy(acc_sh.at[pl.ds(s * RPT, RPT)],
                    acc_hbm.at[c, pl.ds(s * RPT, RPT)])
    pltpu.sync_copy(den_sh.at[pl.ds(s * RPT, RPT)],
                    den_hbm.at[c, pl.ds(s * RPT, RPT)])


@functools.lru_cache(maxsize=1)
def _edge_pass_fn():
    return pl.kernel(
        _edge_body,
        out_type=(
            jax.ShapeDtypeStruct((2, NPAD, D), f32),
            jax.ShapeDtypeStruct((2, NPAD), f32),
        ),
        mesh=plsc.VectorSubcoreMesh(core_axis_name="c", subcore_axis_name="s"),
        compiler_params=pltpu.CompilerParams(needs_layout_passes=False),
        scratch_types=(
            (pltpu.VMEM((NPAD,), f32),) * 2
            + (pltpu.VMEM((B,), i32),) * 3
            + (pltpu.VMEM((B,), i32),) * 3
            + (pltpu.VMEM((B,), f32),) * 3
            + (pltpu.VMEM((B, D), f32),) * 3
            + (
                pltpu.VMEM_SHARED((NPAD, D), f32),
                pltpu.VMEM_SHARED((NPAD,), f32),
            )
            + (pltpu.SemaphoreType.DMA,) * 9
        ),
    )


def _edge_pass(h, asv, adv, src, dst):
    return _edge_pass_fn()(h, asv, adv, src, dst)


def kernel(x, edge_index, graph_pool, W1, a1s, a1d, b1, g1, be1,
           W2, a2s, a2d, b2, g2, be2, W3, a3s, a3d, b3):
    ei = edge_index.astype(i32)
    loop = jnp.arange(N, dtype=i32)
    npad_e = EPAD - ET
    pad_idx = N + (jnp.arange(npad_e, dtype=i32) % 128)
    src = jnp.concatenate([ei[0], loop, pad_idx])
    dst = jnp.concatenate([ei[1], loop, pad_idx])

    x_p = jnp.zeros((NPAD, D), f32).at[:N].set(x)
    gp_p = jnp.zeros((G, NPAD), f32).at[:, :N].set(graph_pool)

    h1, as1, ad1 = _tc_pre(x_p, W1, a1s, a1d)
    acc1, den1 = _edge_pass(h1, as1, ad1, src, dst)
    h2, as2, ad2 = _tc_mid(acc1, den1, b1, g1, be1, W2, a2s, a2d)
    acc2, den2 = _edge_pass(h2, as2, ad2, src, dst)
    h3, as3, ad3 = _tc_mid(acc2, den2, b2, g2, be2, W3, a3s, a3d)
    acc3, den3 = _edge_pass(h3, as3, ad3, src, dst)
    pooled, hn = _tc_fin(acc3, den3, b3, gp_p)
    return (pooled, hn[:N])
